# Initial kernel scaffold; baseline (speedup 1.0000x reference)
import jax
import jax.numpy as jnp
from jax.experimental import pallas as pl


def kernel(attn_s):
    raise NotImplementedError("write your pallas kernel here")


# speedup vs baseline: 3.5268x; 3.5268x over previous
import functools

import jax
import jax.numpy as jnp
from jax import lax
from jax.experimental import pallas as pl
from jax.experimental.pallas import tpu as pltpu
from jax.experimental.pallas import tpu_sc as plsc

B, M, T, S = 2, 16, 512, 2048
P = T * S
NC, NS, L = 2, 16, 16
NW = NC * NS
SPAN = (B * P) // NW
CHUNK = 1024
NCH = SPAN // CHUNK
R = NCH // 2
TOPK = 5
EPS = 0.0001


def _pruned_network():
    ces = []

    def merge(lo, hi, r):
        step = r * 2
        if step < hi - lo:
            merge(lo, hi, step)
            merge(lo + r, hi, step)
            for i in range(lo + r, hi - r, step):
                ces.append((i, i + r))
        else:
            ces.append((lo, lo + r))

    def sort(lo, hi):
        if hi - lo >= 1:
            mid = lo + (hi - lo) // 2
            sort(lo, mid)
            sort(mid + 1, hi)
            merge(lo, hi, 1)

    sort(0, M - 1)
    target = M - TOPK
    needed = {target}
    kept = []
    for (i, j) in reversed(ces):
        ni, nj = i in needed, j in needed
        if ni or nj:
            kept.append((i, j, ni, nj))
            needed.add(i)
            needed.add(j)
    kept.reverse()
    return kept, target


_KEPT, _TARGET = _pruned_network()


def _sc_body(x_hbm, o_hbm, ib0, ib1, ob0, ob1, is0, is1, os0, os1):
    wid = lax.axis_index("s") * NC + lax.axis_index("c")
    b = wid // NS
    base = (wid % NS) * SPAN

    def in_copy(t, buf, sem):
        return pltpu.make_async_copy(
            x_hbm.at[b, :, pl.ds(base + t * CHUNK, CHUNK)], buf, sem)

    def out_copy(t, buf, sem):
        return pltpu.make_async_copy(
            buf, o_hbm.at[b, :, pl.ds(base + t * CHUNK, CHUNK)], sem)

    def compute(src, dst):
        def grp(g, carry):
            sl = pl.ds(g * L, L)
            xs = [src[m, sl] for m in range(M)]
            w = list(xs)
            for (i, j, ni, nj) in _KEPT:
                a, c = w[i], w[j]
                if ni:
                    w[i] = jnp.minimum(a, c)
                if nj:
                    w[j] = jnp.maximum(a, c)
            thr = w[_TARGET] - EPS
            for m in range(M):
                dst[m, sl] = jnp.where(xs[m] > thr, xs[m], 0.0)
            return carry

        lax.fori_loop(0, CHUNK // L, grp, 0)

    def round_slot(t, ibuf, obuf, isem, osem, first, last):
        in_copy(t, ibuf, isem).wait()
        if not first:
            out_copy(t - 2, obuf, osem).wait()
        compute(ibuf, obuf)
        if not last:
            in_copy(t + 2, ibuf, isem).start()
        out_copy(t, obuf, osem).start()

    in_copy(0, ib0, is0).start()
    in_copy(1, ib1, is1).start()

    round_slot(0, ib0, ob0, is0, os0, True, False)
    round_slot(1, ib1, ob1, is1, os1, True, False)

    def mid(r, carry):
        round_slot(2 * r, ib0, ob0, is0, os0, False, False)
        round_slot(2 * r + 1, ib1, ob1, is1, os1, False, False)
        return carry

    lax.fori_loop(1, R - 1, mid, 0)

    round_slot(NCH - 2, ib0, ob0, is0, os0, False, True)
    round_slot(NCH - 1, ib1, ob1, is1, os1, False, True)

    out_copy(NCH - 2, ob0, os0).wait()
    out_copy(NCH - 1, ob1, os1).wait()


@functools.partial(
    pl.kernel,
    out_type=jax.ShapeDtypeStruct((B, M, P), jnp.float32),
    mesh=plsc.VectorSubcoreMesh(core_axis_name="c", subcore_axis_name="s"),
    scratch_types=[
        pltpu.VMEM((M, CHUNK), jnp.float32),
        pltpu.VMEM((M, CHUNK), jnp.float32),
        pltpu.VMEM((M, CHUNK), jnp.float32),
        pltpu.VMEM((M, CHUNK), jnp.float32),
        pltpu.SemaphoreType.DMA,
        pltpu.SemaphoreType.DMA,
        pltpu.SemaphoreType.DMA,
        pltpu.SemaphoreType.DMA,
    ],
)
def _sc_topk_mask(x_hbm, o_hbm, ib0, ib1, ob0, ob1, is0, is1, os0, os1):
    _sc_body(x_hbm, o_hbm, ib0, ib1, ob0, ob1, is0, is1, os0, os1)


def kernel(attn_s):
    x = attn_s.reshape(B, M, P)
    out = _sc_topk_mask(x)
    return out.reshape(B, M, T, S)


# speedup vs baseline: 8.7466x; 2.4800x over previous
import functools

import jax
import jax.numpy as jnp
from jax import lax
from jax.experimental import pallas as pl
from jax.experimental.pallas import tpu as pltpu
from jax.experimental.pallas import tpu_sc as plsc

B, M, T, S = 2, 16, 512, 2048
P = T * S
NC, NS, L = 2, 16, 16
NW = NC * NS
SPAN = (B * P) // NW
CHUNK = 1024
NCH = SPAN // CHUNK
R = NCH // 2
TOPK = 5
EPS = 0.0001


def _pruned_network():
    ces = []

    def merge(lo, hi, r):
        step = r * 2
        if step < hi - lo:
            merge(lo, hi, step)
            merge(lo + r, hi, step)
            for i in range(lo + r, hi - r, step):
                ces.append((i, i + r))
        else:
            ces.append((lo, lo + r))

    def sort(lo, hi):
        if hi - lo >= 1:
            mid = lo + (hi - lo) // 2
            sort(lo, mid)
            sort(mid + 1, hi)
            merge(lo, hi, 1)

    sort(0, M - 1)
    target = M - TOPK
    needed = {target}
    kept = []
    for (i, j) in reversed(ces):
        ni, nj = i in needed, j in needed
        if ni or nj:
            kept.append((i, j, ni, nj))
            needed.add(i)
            needed.add(j)
    kept.reverse()
    return kept, target


_KEPT, _TARGET = _pruned_network()


ROWS_PER_W = SPAN // S
CPR = S // CHUNK


def _sc_body(x_hbm, o_hbm, ib0, ib1, ob0, ob1, is0, is1, os0, os1):
    wid = lax.axis_index("s") * NC + lax.axis_index("c")
    b = wid // NS
    row0 = (wid % NS) * ROWS_PER_W

    def in_copy(t, buf, sem):
        return pltpu.make_async_copy(
            x_hbm.at[b, :, row0 + t // CPR, pl.ds((t % CPR) * CHUNK, CHUNK)],
            buf, sem)

    def out_copy(t, buf, sem):
        return pltpu.make_async_copy(
            buf,
            o_hbm.at[b, :, row0 + t // CPR, pl.ds((t % CPR) * CHUNK, CHUNK)],
            sem)

    def compute(src, dst):
        def grp(g, carry):
            sl = pl.ds(g * L, L)
            xs = [src[m, sl] for m in range(M)]
            w = list(xs)
            for (i, j, ni, nj) in _KEPT:
                a, c = w[i], w[j]
                if ni:
                    w[i] = jnp.minimum(a, c)
                if nj:
                    w[j] = jnp.maximum(a, c)
            thr = w[_TARGET] - EPS
            for m in range(M):
                dst[m, sl] = jnp.where(xs[m] > thr, xs[m], 0.0)
            return carry

        lax.fori_loop(0, CHUNK // L, grp, 0)

    def round_slot(t, ibuf, obuf, isem, osem, first, last):
        in_copy(t, ibuf, isem).wait()
        if not first:
            out_copy(t - 2, obuf, osem).wait()
        compute(ibuf, obuf)
        if not last:
            in_copy(t + 2, ibuf, isem).start()
        out_copy(t, obuf, osem).start()

    in_copy(0, ib0, is0).start()
    in_copy(1, ib1, is1).start()

    round_slot(0, ib0, ob0, is0, os0, True, False)
    round_slot(1, ib1, ob1, is1, os1, True, False)

    def mid(r, carry):
        round_slot(2 * r, ib0, ob0, is0, os0, False, False)
        round_slot(2 * r + 1, ib1, ob1, is1, os1, False, False)
        return carry

    lax.fori_loop(1, R - 1, mid, 0)

    round_slot(NCH - 2, ib0, ob0, is0, os0, False, True)
    round_slot(NCH - 1, ib1, ob1, is1, os1, False, True)

    out_copy(NCH - 2, ob0, os0).wait()
    out_copy(NCH - 1, ob1, os1).wait()


@functools.partial(
    pl.kernel,
    out_type=jax.ShapeDtypeStruct((B, M, T, S), jnp.float32),
    mesh=plsc.VectorSubcoreMesh(core_axis_name="c", subcore_axis_name="s"),
    compiler_params=pltpu.CompilerParams(use_tc_tiling_on_sc=True),
    scratch_types=[
        pltpu.VMEM((M, CHUNK), jnp.float32),
        pltpu.VMEM((M, CHUNK), jnp.float32),
        pltpu.VMEM((M, CHUNK), jnp.float32),
        pltpu.VMEM((M, CHUNK), jnp.float32),
        pltpu.SemaphoreType.DMA,
        pltpu.SemaphoreType.DMA,
        pltpu.SemaphoreType.DMA,
        pltpu.SemaphoreType.DMA,
    ],
)
def _sc_topk_mask(x_hbm, o_hbm, ib0, ib1, ob0, ob1, is0, is1, os0, os1):
    _sc_body(x_hbm, o_hbm, ib0, ib1, ob0, ob1, is0, is1, os0, os1)


def kernel(attn_s):
    return _sc_topk_mask(attn_s)


# speedup vs baseline: 9.7540x; 1.1152x over previous
import functools

import jax
import jax.numpy as jnp
from jax import lax
from jax.experimental import pallas as pl
from jax.experimental.pallas import tpu as pltpu
from jax.experimental.pallas import tpu_sc as plsc

B, M, T, S = 2, 16, 512, 2048
NC, NS, L = 2, 16, 16
NW = NC * NS
ROWS_PER_W = (B * T) // NW
BR, BC = 8, 128
NRB = ROWS_PER_W // BR
NCB = S // BC
NCHUNK = NRB * NCB
NSLOT = 4
GRP = (BR * BC) // L
EPS = 0.0001

_SORT4 = [(0, 1), (2, 3), (0, 2), (1, 3), (1, 2)]


def _pruned_merge44():
    ces = []

    def merge(lo, hi, r):
        step = r * 2
        if step < hi - lo:
            merge(lo, hi, step)
            merge(lo + r, hi, step)
            for i in range(lo + r, hi - r, step):
                ces.append((i, i + r))
        else:
            ces.append((lo, lo + r))

    merge(0, 7, 1)
    needed = set(range(3, 8))
    kept = []
    for (i, j) in reversed(ces):
        ni, nj = i in needed, j in needed
        if ni or nj:
            kept.append((i, j, ni, nj))
            needed.add(i)
            needed.add(j)
    kept.reverse()
    return kept


_MERGE44 = _pruned_merge44()


def _fifth_largest(xs):
    w = list(xs)
    for g in range(4):
        for (i, j) in _SORT4:
            a, c = w[4 * g + i], w[4 * g + j]
            w[4 * g + i] = jnp.minimum(a, c)
            w[4 * g + j] = jnp.maximum(a, c)

    def merge_top5(sub):
        v = list(sub)
        for (i, j, ni, nj) in _MERGE44:
            a, c = v[i], v[j]
            if ni:
                v[i] = jnp.minimum(a, c)
            if nj:
                v[j] = jnp.maximum(a, c)
        return v[3:8]

    a5 = merge_top5(w[0:8])
    b5 = merge_top5(w[8:16])
    t = jnp.maximum(a5[4], b5[0])
    for i in range(1, 5):
        t = jnp.minimum(t, jnp.maximum(a5[4 - i], b5[i]))
    return t


def _sc_body(x_hbm, o_hbm, bufs, isems, osems):
    wid = lax.axis_index("s") * NC + lax.axis_index("c")
    row_g = wid * ROWS_PER_W
    b = row_g // T
    row0 = row_g % T

    def in_copy(t, buf, sem):
        rb, cb = t // NCB, t % NCB
        return pltpu.make_async_copy(
            x_hbm.at[b, :, pl.ds(row0 + rb * BR, BR), pl.ds(cb * BC, BC)],
            buf, sem)

    def out_copy(t, buf, sem):
        rb, cb = t // NCB, t % NCB
        return pltpu.make_async_copy(
            buf,
            o_hbm.at[b, :, pl.ds(row0 + rb * BR, BR), pl.ds(cb * BC, BC)],
            sem)

    def compute(buf):
        def grp(idx, carry):
            rr = idx // (BC // L)
            sl = pl.ds((idx % (BC // L)) * L, L)
            xs = [buf[m, rr, sl] for m in range(M)]
            thr = _fifth_largest(xs) - EPS
            for m in range(M):
                buf[m, rr, sl] = jnp.where(xs[m] > thr, xs[m], 0.0)
            return carry

        lax.fori_loop(0, GRP, grp, 0)

    def chunk(t, s):
        sp = (s + 1) % NSLOT
        tn = t + 1

        @pl.when(tn < NCHUNK)
        def _prefetch():
            @pl.when(tn >= NSLOT)
            def _drain():
                out_copy(tn - NSLOT, bufs[sp], osems[sp]).wait()

            in_copy(tn, bufs[sp], isems[sp]).start()

        in_copy(t, bufs[s], isems[s]).wait()
        compute(bufs[s])
        out_copy(t, bufs[s], osems[s]).start()

    in_copy(0, bufs[0], isems[0]).start()

    def superstep(ss, carry):
        for s in range(NSLOT):
            chunk(ss * NSLOT + s, s)
        return carry

    lax.fori_loop(0, NCHUNK // NSLOT, superstep, 0)

    for k in range(NSLOT):
        t = NCHUNK - NSLOT + k
        out_copy(t, bufs[t % NSLOT], osems[t % NSLOT]).wait()


@functools.partial(
    pl.kernel,
    out_type=jax.ShapeDtypeStruct((B, M, T, S), jnp.float32),
    mesh=plsc.VectorSubcoreMesh(core_axis_name="c", subcore_axis_name="s"),
    compiler_params=pltpu.CompilerParams(use_tc_tiling_on_sc=True),
    scratch_types=(
        [pltpu.VMEM((M, BR, BC), jnp.float32) for _ in range(NSLOT)]
        + [pltpu.SemaphoreType.DMA for _ in range(2 * NSLOT)]
    ),
)
def _sc_topk_mask(x_hbm, o_hbm, b0, b1, b2, b3, i0, i1, i2, i3, o0, o1, o2, o3):
    _sc_body(x_hbm, o_hbm, [b0, b1, b2, b3], [i0, i1, i2, i3],
             [o0, o1, o2, o3])


def kernel(attn_s):
    return _sc_topk_mask(attn_s)
